# speedup vs baseline: 3.2216x; 3.2216x over previous
import functools

import jax
import jax.numpy as jnp
from jax import lax
from jax.experimental import pallas as pl
from jax.experimental.pallas import tpu as pltpu
from jax.experimental.pallas import tpu_sc as plsc

_N = 10000
_DEG = 32
_D = 128
_NUM = 2
_R = _N * _NUM
_G = _R * _DEG

_SC_C = 80
_SC_K = 5
_SC_GRP = _SC_C * _SC_K
_NW = 32

_BLK = 200


def _sc_gather(table, idx):
  g_total = idx.shape[0]
  d = table.shape[1]
  per_w = g_total // _NW
  ngrp = per_w // _SC_GRP
  assert per_w % _SC_GRP == 0

  mesh = plsc.VectorSubcoreMesh(core_axis_name="c", subcore_axis_name="s")

  @functools.partial(
      pl.kernel,
      out_type=jax.ShapeDtypeStruct((g_total, d), jnp.float32),
      mesh=mesh,
      scratch_types=[
          pltpu.VMEM((_SC_GRP,), jnp.int32),
          pltpu.VMEM((_SC_GRP, d), jnp.float32),
          pltpu.SemaphoreType.DMA,
      ],
  )
  def gather_k(table_hbm, idx_hbm, out_hbm, idx_v, rows_v, gsem):
    wid = lax.axis_index("s") * 2 + lax.axis_index("c")
    base = wid * per_w

    def group(gi, carry):
      gbase = base + gi * _SC_GRP
      pltpu.sync_copy(idx_hbm.at[pl.ds(gbase, _SC_GRP)], idx_v)
      copies = []
      for j in range(_SC_K):
        copies.append(
            pltpu.async_copy(
                table_hbm.at[idx_v.at[pl.ds(j * _SC_C, _SC_C)]],
                rows_v.at[pl.ds(j * _SC_C, _SC_C)],
                gsem,
            ))
      for cp in copies:
        cp.wait()
      pltpu.sync_copy(rows_v, out_hbm.at[pl.ds(gbase, _SC_GRP)])
      return carry

    lax.fori_loop(0, ngrp, group, 0)

  return gather_k(table, idx)


def _tc_layer(h, m, ew, w_in, w_hh, bias, w_self, w_neigh, b_neigh):
  nblk = _R // _BLK

  def body(h_ref, m_ref, ew_ref, win_ref, whh_ref, b_ref, ws_ref, wn_ref,
           bn_ref, out_ref):
    h0 = h_ref[...]
    mm = m_ref[...] * ew_ref[...][:, :, None]
    win = win_ref[...]
    whh = whh_ref[...]
    b = b_ref[...]
    ht = jnp.zeros((_BLK, _D), jnp.float32)
    ct = jnp.zeros((_BLK, _D), jnp.float32)
    for t in range(_DEG):
      g = (jnp.dot(mm[:, t, :], win, preferred_element_type=jnp.float32)
           + jnp.dot(ht, whh, preferred_element_type=jnp.float32) + b)
      ig = jax.nn.sigmoid(g[:, :_D])
      fg = jax.nn.sigmoid(g[:, _D:2 * _D])
      gg = jnp.tanh(g[:, 2 * _D:3 * _D])
      og = jax.nn.sigmoid(g[:, 3 * _D:])
      ct = fg * ct + ig * gg
      ht = og * jnp.tanh(ct)
    out_ref[...] = (jnp.dot(h0, ws_ref[...], preferred_element_type=jnp.float32)
                    + jnp.dot(ht, wn_ref[...], preferred_element_type=jnp.float32)
                    + bn_ref[...])

  full = lambda i: (0, 0)
  return pl.pallas_call(
      body,
      grid=(nblk,),
      in_specs=[
          pl.BlockSpec((_BLK, _D), lambda i: (i, 0)),
          pl.BlockSpec((_BLK, _DEG, _D), lambda i: (i, 0, 0)),
          pl.BlockSpec((_BLK, _DEG), lambda i: (i, 0)),
          pl.BlockSpec((_D, 4 * _D), full),
          pl.BlockSpec((_D, 4 * _D), full),
          pl.BlockSpec((1, 4 * _D), full),
          pl.BlockSpec((_D, _D), full),
          pl.BlockSpec((_D, _D), full),
          pl.BlockSpec((1, _D), full),
      ],
      out_specs=pl.BlockSpec((_BLK, _D), lambda i: (i, 0)),
      out_shape=jax.ShapeDtypeStruct((_R, _D), jnp.float32),
  )(h, m, ew, w_in, w_hh, bias, w_self, w_neigh, b_neigh)


def kernel(x, nbr1, nbr2, nbr3, ew1, ew2, ew3,
           Wih1, Whh1, bih1, bhh1, Wself1, Wneigh1, bneigh1,
           Wih2, Whh2, bih2, bhh2, Wself2, Wneigh2, bneigh2,
           Wih3, Whh3, bih3, bhh3, Wself3, Wneigh3, bneigh3):
  h = x.reshape(_R, _D)
  coff = jnp.arange(_NUM, dtype=jnp.int32)[None, :, None]

  layers = []
  for nbr, ew, Wih, Whh, bih, bhh, Wself, Wneigh, bneigh in (
      (nbr1, ew1, Wih1, Whh1, bih1, bhh1, Wself1, Wneigh1, bneigh1),
      (nbr2, ew2, Wih2, Whh2, bih2, bhh2, Wself2, Wneigh2, bneigh2),
      (nbr3, ew3, Wih3, Whh3, bih3, bhh3, Wself3, Wneigh3, bneigh3)):
    idx = (nbr[:, None, :] * _NUM + coff).reshape(_G)
    ew_b = jnp.broadcast_to(ew[:, None, :], (_N, _NUM, _DEG)).reshape(_R, _DEG)
    layers.append((idx, ew_b, Wih.T, Whh.T, (bih + bhh)[None, :],
                   Wself.T, Wneigh.T, bneigh[None, :]))

  for idx, ew_b, w_in, w_hh, bias, w_self, w_neigh, b_neigh in layers:
    m = _sc_gather(h, idx).reshape(_R, _DEG, _D)
    h = _tc_layer(h, m, ew_b, w_in, w_hh, bias, w_self, w_neigh, b_neigh)

  return h.reshape(_N, _NUM, _D)
